# Initial kernel scaffold; baseline (speedup 1.0000x reference)
import jax
import jax.numpy as jnp
from jax.experimental import pallas as pl


def kernel(x, pe):
    raise NotImplementedError("write your pallas kernel here")


# speedup vs baseline: 3.4125x; 3.4125x over previous
import functools

import jax
import jax.numpy as jnp
from jax import lax
from jax.experimental import pallas as pl
from jax.experimental.pallas import tpu as pltpu
from jax.experimental.pallas import tpu_sc as plsc

_NUM_CORES = 2
_NUM_SUBCORES = 16
_NUM_WORKERS = _NUM_CORES * _NUM_SUBCORES

_B = 4
_T = 8192
_H = 1024
_ROWS_PER_WORKER = _T // _NUM_WORKERS
_SUB = 64
_NSUB = _ROWS_PER_WORKER // _SUB
_G = 16
_NGROUPS = _ROWS_PER_WORKER // _G


def _body(x_hbm, pe_hbm, out_hbm, chunk, pe0, x_v):
    wid = lax.axis_index("s") * _NUM_CORES + lax.axis_index("c")
    base = wid * _ROWS_PER_WORKER

    pltpu.sync_copy(pe_hbm.at[pl.ds(0, 1)], pe0)
    pltpu.sync_copy(x_hbm.at[:, pl.ds(base, _ROWS_PER_WORKER)], x_v)

    for sub in range(_NSUB):
        rb = base + sub * _SUB
        pltpu.sync_copy(pe_hbm.at[pl.ds(rb, _SUB)], chunk)
        for b in range(_B):
            pltpu.sync_copy(chunk, out_hbm.at[b, pl.ds(rb, _SUB)])

    for b in range(_B):
        def group(g, carry, b=b):
            xv = x_v[b, pl.ds(g * _G, _G)]
            hit = jnp.any(xv == 0)

            @pl.when(hit)
            def _():
                for r in range(_G):
                    @pl.when(xv[r] == 0)
                    def _(b=b, r=r):
                        pltpu.sync_copy(
                            pe0, out_hbm.at[b, pl.ds(base + g * _G + r, 1)])

            return carry

        lax.fori_loop(0, _NGROUPS, group, 0)


@jax.jit
def kernel(x, pe):
    mesh = plsc.VectorSubcoreMesh(
        core_axis_name="c", subcore_axis_name="s",
        num_cores=_NUM_CORES, num_subcores=_NUM_SUBCORES)
    run = pl.kernel(
        _body,
        out_type=jax.ShapeDtypeStruct((_B, _T, _H), jnp.float32),
        mesh=mesh,
        compiler_params=pltpu.CompilerParams(needs_layout_passes=False),
        scratch_types=[
            pltpu.VMEM((_SUB, _H), jnp.float32),
            pltpu.VMEM((1, _H), jnp.float32),
            pltpu.VMEM((_B, _ROWS_PER_WORKER), jnp.int32),
        ],
    )
    return run(x, pe)
